# Initial kernel scaffold; baseline (speedup 1.0000x reference)
import jax
import jax.numpy as jnp
from jax.experimental import pallas as pl


def kernel(inp, edge_index, W, b):
    raise NotImplementedError("write your pallas kernel here")


# speedup vs baseline: 22.5562x; 22.5562x over previous
import functools

import jax
import jax.numpy as jnp
from jax import lax
from jax.experimental import pallas as pl
from jax.experimental.pallas import tpu as pltpu
from jax.experimental.pallas import tpu_sc as plsc

N = 10000
E = 320000
D = 128

NC = 2
NS = 16
NW = NC * NS

B = 128
NCH = -(-E // (NW * B))
E_PAD = NW * NCH * B
PAD_SPREAD = 128

N_PAD = 10240
STRIPE = N_PAD // NS
BLK = 512


def _sc_mesh():
    return plsc.VectorSubcoreMesh(core_axis_name="c", subcore_axis_name="s")


@functools.partial(
    pl.kernel,
    out_type=jax.ShapeDtypeStruct((NC, N_PAD), jnp.float32),
    mesh=_sc_mesh(),
    scratch_types=[
        pltpu.VMEM((NCH, B), jnp.int32),
        pltpu.VMEM((B,), jnp.float32),
        pltpu.VMEM((STRIPE,), jnp.float32),
        pltpu.VMEM_SHARED((N_PAD,), jnp.float32),
    ],
)
def _deg_kernel(col_hbm, deg_out, idx_v, ones_v, zer_v, deg_sh):
    c = lax.axis_index("c")
    s = lax.axis_index("s")
    wid = s * NC + c
    pltpu.sync_copy(col_hbm.at[wid], idx_v)
    ones16 = jnp.ones((16,), jnp.float32)
    zero16 = jnp.zeros((16,), jnp.float32)
    for i in range(B // 16):
        ones_v[pl.ds(i * 16, 16)] = ones16
    for i in range(STRIPE // 16):
        zer_v[pl.ds(i * 16, 16)] = zero16
    pltpu.sync_copy(zer_v, deg_sh.at[pl.ds(s * STRIPE, STRIPE)])
    plsc.subcore_barrier()

    def body(j, carry):
        pltpu.sync_copy(ones_v, deg_sh.at[idx_v.at[j]], add=True)
        return carry

    lax.fori_loop(0, NCH, body, 0)
    plsc.subcore_barrier()
    pltpu.sync_copy(deg_sh.at[pl.ds(s * STRIPE, STRIPE)],
                    deg_out.at[c, pl.ds(s * STRIPE, STRIPE)])


@functools.partial(
    pl.kernel,
    out_type=jax.ShapeDtypeStruct((NC, N_PAD, D), jnp.float32),
    mesh=_sc_mesh(),
    scratch_types=[
        pltpu.VMEM((NCH, B), jnp.int32),
        pltpu.VMEM((NCH, B), jnp.int32),
        pltpu.VMEM((B, D), jnp.float32),
        pltpu.VMEM_SHARED((N_PAD, D), jnp.float32),
        pltpu.SemaphoreType.DMA,
    ],
)
def _prop_kernel(y_hbm, row_hbm, col_hbm, out_hbm, idxr_v, idxc_v, buf_v,
                 z_sh, sem):
    c = lax.axis_index("c")
    s = lax.axis_index("s")
    wid = s * NC + c
    pltpu.sync_copy(row_hbm.at[wid], idxr_v)
    pltpu.sync_copy(col_hbm.at[wid], idxc_v)

    zero16 = jnp.zeros((16,), jnp.float32)

    def zb(b, carry):
        for jj in range(D // 16):
            buf_v[b, pl.ds(jj * 16, 16)] = zero16
        return carry

    lax.fori_loop(0, B, zb, 0)
    for k in range(STRIPE // B):
        pltpu.sync_copy(buf_v, z_sh.at[pl.ds(s * STRIPE + k * B, B)])
    plsc.subcore_barrier()

    def body(j, carry):
        pltpu.async_copy(y_hbm.at[idxr_v.at[j]], buf_v, sem).wait()
        pltpu.sync_copy(buf_v, z_sh.at[idxc_v.at[j]], add=True)
        return carry

    lax.fori_loop(0, NCH, body, 0)
    plsc.subcore_barrier()
    pltpu.sync_copy(z_sh.at[pl.ds(s * STRIPE, STRIPE)],
                    out_hbm.at[c, pl.ds(s * STRIPE, STRIPE)])


def _deg_block(degp_ref):
    deg = degp_ref[0, :] + degp_ref[1, :] + 1.0
    return deg


def _scale0_body(degp_ref, x_ref, y_ref):
    dis = lax.rsqrt(_deg_block(degp_ref))
    y_ref[...] = x_ref[...] * dis[:, None]


def _combine_body(degp_ref, y0_ref, p_ref, y1_ref):
    inv = 1.0 / _deg_block(degp_ref)
    z = y0_ref[...] + p_ref[0] + p_ref[1]
    y1_ref[...] = z * inv[:, None]


def _final_body(degp_ref, y1_ref, q_ref, w_ref, b_ref, o_ref):
    dis = lax.rsqrt(_deg_block(degp_ref))
    z = y1_ref[...] + q_ref[0] + q_ref[1]
    t = z * dis[:, None]
    o_ref[...] = jnp.dot(t, w_ref[...],
                         preferred_element_type=jnp.float32) + b_ref[...]


_G = N_PAD // BLK

_degp_spec = pl.BlockSpec((NC, BLK), lambda i: (0, i))
_rows_spec = pl.BlockSpec((BLK, D), lambda i: (i, 0))
_pair_spec = pl.BlockSpec((NC, BLK, D), lambda i: (0, i, 0))


def _scale0(degp, x_pad):
    return pl.pallas_call(
        _scale0_body,
        grid=(_G,),
        in_specs=[_degp_spec, _rows_spec],
        out_specs=_rows_spec,
        out_shape=jax.ShapeDtypeStruct((N_PAD, D), jnp.float32),
    )(degp, x_pad)


def _combine(degp, y0, p):
    return pl.pallas_call(
        _combine_body,
        grid=(_G,),
        in_specs=[_degp_spec, _rows_spec, _pair_spec],
        out_specs=_rows_spec,
        out_shape=jax.ShapeDtypeStruct((N_PAD, D), jnp.float32),
    )(degp, y0, p)


def _final(degp, y1, q, w, b2):
    return pl.pallas_call(
        _final_body,
        grid=(_G,),
        in_specs=[
            _degp_spec, _rows_spec, _pair_spec,
            pl.BlockSpec((D, D), lambda i: (0, 0)),
            pl.BlockSpec((1, D), lambda i: (0, 0)),
        ],
        out_specs=_rows_spec,
        out_shape=jax.ShapeDtypeStruct((N_PAD, D), jnp.float32),
    )(degp, y1, q, w, b2)


def kernel(inp, edge_index, W, b):
    row = edge_index[0]
    col = edge_index[1]
    pad = jnp.arange(E_PAD - E, dtype=jnp.int32) % PAD_SPREAD + N
    rowp = jnp.concatenate([row, pad]).reshape(NW, NCH, B)
    colp = jnp.concatenate([col, pad]).reshape(NW, NCH, B)
    x_pad = jnp.pad(inp, ((0, N_PAD - N), (0, 0)))

    degp = _deg_kernel(colp)
    y0 = _scale0(degp, x_pad)
    p = _prop_kernel(y0, rowp, colp)
    y1 = _combine(degp, y0, p)
    q = _prop_kernel(y1, rowp, colp)
    out_full = _final(degp, y1, q, W, b.reshape(1, D))
    return out_full[:N]


# speedup vs baseline: 28.8799x; 1.2804x over previous
import functools

import jax
import jax.numpy as jnp
from jax import lax
from jax.experimental import pallas as pl
from jax.experimental.pallas import tpu as pltpu
from jax.experimental.pallas import tpu_sc as plsc

N = 10000
E = 320000
D = 128

NC = 2
NS = 16
NW = NC * NS

B = 128
GH = 64
NRB = 4
NCH = 80
E_PAD = NW * NCH * B
PAD_SPREAD = 128

N_PAD = 10240
STRIPE = N_PAD // NS
BLK = 512


def _sc_mesh():
    return plsc.VectorSubcoreMesh(core_axis_name="c", subcore_axis_name="s")


@functools.partial(
    pl.kernel,
    out_type=jax.ShapeDtypeStruct((NC, N_PAD), jnp.float32),
    mesh=_sc_mesh(),
    scratch_types=[
        pltpu.VMEM((NCH, B), jnp.int32),
        pltpu.VMEM((B,), jnp.float32),
        pltpu.VMEM((STRIPE,), jnp.float32),
        pltpu.VMEM_SHARED((N_PAD,), jnp.float32),
    ],
)
def _deg_kernel(col_hbm, deg_out, idx_v, ones_v, zer_v, deg_sh):
    c = lax.axis_index("c")
    s = lax.axis_index("s")
    wid = s * NC + c
    pltpu.sync_copy(col_hbm.at[wid], idx_v)
    ones16 = jnp.ones((16,), jnp.float32)
    zero16 = jnp.zeros((16,), jnp.float32)
    for i in range(B // 16):
        ones_v[pl.ds(i * 16, 16)] = ones16
    for i in range(STRIPE // 16):
        zer_v[pl.ds(i * 16, 16)] = zero16
    pltpu.sync_copy(zer_v, deg_sh.at[pl.ds(s * STRIPE, STRIPE)])
    plsc.subcore_barrier()

    def body(j, carry):
        pltpu.sync_copy(ones_v, deg_sh.at[idx_v.at[j]], add=True)
        return carry

    lax.fori_loop(0, NCH, body, 0)
    plsc.subcore_barrier()
    pltpu.sync_copy(deg_sh.at[pl.ds(s * STRIPE, STRIPE)],
                    deg_out.at[c, pl.ds(s * STRIPE, STRIPE)])


@functools.partial(
    pl.kernel,
    out_type=jax.ShapeDtypeStruct((NC, N_PAD, D), jnp.float32),
    mesh=_sc_mesh(),
    scratch_types=[
        pltpu.VMEM((NCH, B), jnp.int32),
        pltpu.VMEM((NRB, B), jnp.int32),
        pltpu.VMEM((2, B, D), jnp.float32),
        pltpu.VMEM_SHARED((N_PAD, D), jnp.float32),
        pltpu.SemaphoreType.DMA((4,)),
        pltpu.SemaphoreType.DMA((2,)),
        pltpu.SemaphoreType.DMA((NRB,)),
    ],
)
def _prop_kernel(y_hbm, row_hbm, col_hbm, out_hbm, idxc_v, rowb_v, buf_v,
                 z_sh, gsem, ssem, rsem):
    c = lax.axis_index("c")
    s = lax.axis_index("s")
    wid = s * NC + c
    pltpu.sync_copy(col_hbm.at[wid], idxc_v)

    zero16 = jnp.zeros((16,), jnp.float32)

    def zb(bi, carry):
        for jj in range(D // 16):
            buf_v[0, bi, pl.ds(jj * 16, 16)] = zero16
        return carry

    lax.fori_loop(0, B, zb, 0)
    for k in range(STRIPE // B):
        pltpu.sync_copy(buf_v.at[0], z_sh.at[pl.ds(s * STRIPE + k * B, B)])
    plsc.subcore_barrier()

    def rload(k, rs):
        pltpu.async_copy(row_hbm.at[wid, k], rowb_v.at[rs], rsem.at[rs])

    def rwait(k, rs):
        pltpu.make_async_copy(row_hbm.at[wid, k], rowb_v.at[rs],
                              rsem.at[rs]).wait()

    def _gparts(rs, db, h):
        idx = rowb_v.at[rs, pl.ds(h * GH, GH)]
        dst = buf_v.at[db, pl.ds(h * GH, GH)]
        sem = gsem.at[db * 2 + h]
        return y_hbm.at[idx], dst, sem

    def gissue(rs, db, h):
        src, dst, sem = _gparts(rs, db, h)
        pltpu.async_copy(src, dst, sem)

    def gwait(rs, db, h):
        src, dst, sem = _gparts(rs, db, h)
        pltpu.make_async_copy(src, dst, sem).wait()

    def sissue(k, db):
        pltpu.async_copy(buf_v.at[db], z_sh.at[idxc_v.at[k]],
                         ssem.at[db], add=True)

    def swait(k, db):
        pltpu.make_async_copy(buf_v.at[db], z_sh.at[idxc_v.at[k]],
                              ssem.at[db]).wait()

    def step(k, j4, do_swait=True, do_next=True, do_rload=True):
        db = j4 % 2
        sissue(k, db)
        if do_swait:
            swait(k - 1, 1 - db)
        if do_next:
            rwait(k + 1, (j4 + 1) % NRB)
            gissue((j4 + 1) % NRB, 1 - db, 0)
            gissue((j4 + 1) % NRB, 1 - db, 1)
        if do_rload:
            rload(k + 3, (j4 + 3) % NRB)
        if do_next:
            gwait((j4 + 1) % NRB, 1 - db, 0)
            gwait((j4 + 1) % NRB, 1 - db, 1)

    for k in range(3):
        rload(k, k)
    rwait(0, 0)
    gissue(0, 0, 0)
    gissue(0, 0, 1)
    gwait(0, 0, 0)
    gwait(0, 0, 1)

    for k in range(4):
        step(k, k, do_swait=(k >= 1))

    def body(i, carry):
        k0 = i * 4
        for j in range(4):
            step(k0 + j, j)
        return carry

    lax.fori_loop(1, NCH // 4 - 1, body, 0)

    for k in range(NCH - 4, NCH):
        step(k, k % 4, do_next=(k + 1 < NCH), do_rload=(k + 3 < NCH))
    swait(NCH - 1, (NCH - 1) % 2)
    plsc.subcore_barrier()
    pltpu.sync_copy(z_sh.at[pl.ds(s * STRIPE, STRIPE)],
                    out_hbm.at[c, pl.ds(s * STRIPE, STRIPE)])


def _deg_block(degp_ref):
    deg = degp_ref[0, :] + degp_ref[1, :] + 1.0
    return deg


def _scale0_body(degp_ref, x_ref, y_ref):
    dis = lax.rsqrt(_deg_block(degp_ref))
    y_ref[...] = x_ref[...] * dis[:, None]


def _combine_body(degp_ref, y0_ref, p_ref, y1_ref):
    inv = 1.0 / _deg_block(degp_ref)
    z = y0_ref[...] + p_ref[0] + p_ref[1]
    y1_ref[...] = z * inv[:, None]


def _final_body(degp_ref, y1_ref, q_ref, w_ref, b_ref, o_ref):
    dis = lax.rsqrt(_deg_block(degp_ref))
    z = y1_ref[...] + q_ref[0] + q_ref[1]
    t = z * dis[:, None]
    o_ref[...] = jnp.dot(t, w_ref[...],
                         preferred_element_type=jnp.float32) + b_ref[...]


_G = N_PAD // BLK

_degp_spec = pl.BlockSpec((NC, BLK), lambda i: (0, i))
_rows_spec = pl.BlockSpec((BLK, D), lambda i: (i, 0))
_pair_spec = pl.BlockSpec((NC, BLK, D), lambda i: (0, i, 0))


def _scale0(degp, x_pad):
    return pl.pallas_call(
        _scale0_body,
        grid=(_G,),
        in_specs=[_degp_spec, _rows_spec],
        out_specs=_rows_spec,
        out_shape=jax.ShapeDtypeStruct((N_PAD, D), jnp.float32),
    )(degp, x_pad)


def _combine(degp, y0, p):
    return pl.pallas_call(
        _combine_body,
        grid=(_G,),
        in_specs=[_degp_spec, _rows_spec, _pair_spec],
        out_specs=_rows_spec,
        out_shape=jax.ShapeDtypeStruct((N_PAD, D), jnp.float32),
    )(degp, y0, p)


def _final(degp, y1, q, w, b2):
    return pl.pallas_call(
        _final_body,
        grid=(_G,),
        in_specs=[
            _degp_spec, _rows_spec, _pair_spec,
            pl.BlockSpec((D, D), lambda i: (0, 0)),
            pl.BlockSpec((1, D), lambda i: (0, 0)),
        ],
        out_specs=_rows_spec,
        out_shape=jax.ShapeDtypeStruct((N_PAD, D), jnp.float32),
    )(degp, y1, q, w, b2)


def kernel(inp, edge_index, W, b):
    row = edge_index[0]
    col = edge_index[1]
    pad = jnp.arange(E_PAD - E, dtype=jnp.int32) % PAD_SPREAD + N
    rowp = jnp.concatenate([row, pad]).reshape(NW, NCH, B)
    colp = jnp.concatenate([col, pad]).reshape(NW, NCH, B)
    x_pad = jnp.pad(inp, ((0, N_PAD - N), (0, 0)))

    degp = _deg_kernel(colp)
    y0 = _scale0(degp, x_pad)
    p = _prop_kernel(y0, rowp, colp)
    y1 = _combine(degp, y0, p)
    q = _prop_kernel(y1, rowp, colp)
    out_full = _final(degp, y1, q, W, b.reshape(1, D))
    return out_full[:N]
